# Initial kernel scaffold; baseline (speedup 1.0000x reference)
import jax
import jax.numpy as jnp
from jax.experimental import pallas as pl


def kernel(ten_in, jitter):
    raise NotImplementedError("write your pallas kernel here")


# speedup vs baseline: 143.8430x; 143.8430x over previous
import functools

import jax
import jax.numpy as jnp
from jax import lax
from jax.experimental import pallas as pl
from jax.experimental.pallas import tpu as pltpu
from jax.experimental.pallas import tpu_sc as plsc

B, C, H, W = 2, 96, 224, 224
OH, OW = 448, 448
L = 16
NC, NS = 2, 16
NW = NC * NS
SLICES = B * C
SLICES_PER_TILE = SLICES // NW
RCH = 32
NCH = H // RCH
NQ = SLICES_PER_TILE * NCH
IN_CHUNK = RCH * W
OUT_ROWS = 2 * RCH
OUT_CHUNK = OUT_ROWS * OW
GROUPS = W // L


def _sc_body(in_hbm, jit_hbm, out_hbm, jbuf, inb0, inb1, outb0, outb1,
             zrow, s_in0, s_in1, s_out0, s_out1):
    wid = lax.axis_index("s") * NC + lax.axis_index("c")
    s0 = wid * SLICES_PER_TILE
    b = s0 // C

    pltpu.sync_copy(jit_hbm, jbuf)
    jy0 = jbuf[pl.ds(0, L)]
    jx0 = jbuf[pl.ds(L, L)]
    jy1 = jbuf[pl.ds(2 * L, L)]
    jx1 = jbuf[pl.ds(3 * L, L)]
    bv = jnp.broadcast_to(b, (L,))
    jy = jnp.where(bv == 0, jy0, jy1)
    jx = jnp.where(bv == 0, jx0, jx1)
    half = jnp.full((L,), 0.5, jnp.float32)
    one = jnp.full((L,), 1, jnp.int32)
    two = jnp.full((L,), 2, jnp.int32)
    dyv = jnp.where(jy >= half, two, one)
    dxv = jnp.where(jx >= half, two, one)

    iota = lax.iota(jnp.int32, L)
    wmax = jnp.full((L,), OW - 1, jnp.int32)
    colv = [jnp.minimum(2 * L * g + 2 * iota + dxv, wmax) for g in range(GROUPS)]

    zf = jnp.zeros((L,), jnp.float32)

    def zb(i, _):
        outb0[pl.ds(i * L, L)] = zf
        outb1[pl.ds(i * L, L)] = zf
        return 0
    lax.fori_loop(0, OUT_CHUNK // L, zb, 0)

    def zr(i, _):
        zrow[pl.ds(i * L, L)] = zf
        return 0
    lax.fori_loop(0, OW // L, zr, 0)

    inbs = (inb0, inb1)
    outbs = (outb0, outb1)
    sins = (s_in0, s_in1)
    souts = (s_out0, s_out1)

    def in_copy(q):
        si, ci = q // NCH, q % NCH
        base = (s0 + si) * (H * W) + (ci * RCH) * W
        return pltpu.make_async_copy(
            in_hbm.at[pl.ds(base, IN_CHUNK)], inbs[q % 2], sins[q % 2])

    def out_copy(q):
        si, ci = q // NCH, q % NCH
        rows = OUT_ROWS if ci < NCH - 1 else OUT_ROWS - 1
        dst = (s0 + si) * (OH * OW) + (2 * ci * RCH + 1) * OW
        return pltpu.make_async_copy(
            outbs[q % 2].at[pl.ds(0, rows * OW)],
            out_hbm.at[pl.ds(dst, rows * OW)], souts[q % 2])

    in_copy(0).start()
    for q in range(NQ):
        si, ci = q // NCH, q % NCH
        h0 = ci * RCH
        inb = inbs[q % 2]
        outb = outbs[q % 2]

        in_copy(q).wait()
        if q + 1 < NQ:
            in_copy(q + 1).start()
        if q >= 2:
            out_copy(q - 2).wait()
        if ci == 0:
            pltpu.sync_copy(zrow, out_hbm.at[pl.ds((s0 + si) * (OH * OW), OW)])

        hmax = jnp.full((L,), OH - 1, jnp.int32)

        def row(r, _):
            grow = jnp.minimum(2 * (h0 + r) + dyv, hmax)
            rbase = (grow - (2 * h0 + 1)) * OW
            for g in range(GROUPS):
                x = inb[pl.ds(r * W + g * L, L)]
                plsc.store_scatter(outb, [rbase + colv[g]], x)
            return 0
        lax.fori_loop(0, RCH, row, 0)

        out_copy(q).start()
    out_copy(NQ - 2).wait()
    out_copy(NQ - 1).wait()


@jax.jit
def _zero_upsample_sc(flat_in, jit64):
    mesh = plsc.VectorSubcoreMesh(core_axis_name="c", subcore_axis_name="s")
    f = pl.kernel(
        _sc_body,
        out_type=jax.ShapeDtypeStruct((B * C * OH * OW,), jnp.float32),
        mesh=mesh,
        compiler_params=pltpu.CompilerParams(needs_layout_passes=False),
        scratch_types=[
            pltpu.VMEM((4 * L,), jnp.float32),
            pltpu.VMEM((IN_CHUNK,), jnp.float32),
            pltpu.VMEM((IN_CHUNK,), jnp.float32),
            pltpu.VMEM((OUT_CHUNK,), jnp.float32),
            pltpu.VMEM((OUT_CHUNK,), jnp.float32),
            pltpu.VMEM((OW,), jnp.float32),
            pltpu.SemaphoreType.DMA,
            pltpu.SemaphoreType.DMA,
            pltpu.SemaphoreType.DMA,
            pltpu.SemaphoreType.DMA,
        ],
    )
    return f(flat_in, jit64)


def kernel(ten_in, jitter):
    jit64 = jnp.broadcast_to(
        jitter.astype(jnp.float32).reshape(4, 1), (4, L)).reshape(4 * L)
    flat = _zero_upsample_sc(ten_in.reshape(-1), jit64)
    return flat.reshape(B, C, OH, OW)


# speedup vs baseline: 376.9632x; 2.6207x over previous
import jax
import jax.numpy as jnp
from jax import lax
from jax.experimental import pallas as pl
from jax.experimental.pallas import tpu as pltpu
from jax.experimental.pallas import tpu_sc as plsc

B, C, H, W = 2, 96, 224, 224
OH, OW = 448, 448
L = 16
NC, NS = 2, 16
NW = NC * NS
SLICES = B * C
SPT = SLICES // NW
ORW = 64
NWIN = OH // ORW
NQ = SPT * NWIN
IRW = 40
GROUPS = W // L


def _sc_body(in_hbm, jit_hbm, out_hbm, jbuf, inb0, inb1, outb0, outb1,
             s_in0, s_in1, s_out0, s_out1):
    wid = lax.axis_index("s") * NC + lax.axis_index("c")
    s0 = wid * SPT
    b = s0 // C

    pltpu.sync_copy(jit_hbm, jbuf)
    jy0 = jbuf[pl.ds(0, L)]
    jx0 = jbuf[pl.ds(L, L)]
    jy1 = jbuf[pl.ds(2 * L, L)]
    jx1 = jbuf[pl.ds(3 * L, L)]
    bv = jnp.broadcast_to(b, (L,))
    jy = jnp.where(bv == 0, jy0, jy1)
    jx = jnp.where(bv == 0, jx0, jx1)
    half = jnp.full((L,), 0.5, jnp.float32)
    one = jnp.full((L,), 1, jnp.int32)
    two = jnp.full((L,), 2, jnp.int32)
    dyv = jnp.where(jy >= half, two, one)
    dxv = jnp.where(jx >= half, two, one)

    iota = lax.iota(jnp.int32, L)
    wmax = jnp.full((L,), OW - 1, jnp.int32)
    hmax = jnp.full((L,), OH - 1, jnp.int32)
    zero = jnp.full((L,), 0, jnp.int32)
    rows = jnp.full((L,), ORW, jnp.int32)
    colv = [jnp.minimum(2 * L * g + 2 * iota + dxv, wmax) for g in range(GROUPS)]

    zf = jnp.zeros((L,), jnp.float32)

    for g in range(OW // L):
        def zb(i, _, g=g):
            outb0[i, pl.ds(g * L, L)] = zf
            outb1[i, pl.ds(g * L, L)] = zf
            return 0
        lax.fori_loop(0, ORW, zb, 0)

    inbs = (inb0, inb1)
    outbs = (outb0, outb1)
    sins = (s_in0, s_in1)
    souts = (s_out0, s_out1)

    def in_copy(q):
        si, k = q // NWIN, q % NWIN
        hs = max(32 * k - 8, 0)
        return pltpu.make_async_copy(
            in_hbm.at[s0 + si, pl.ds(hs, IRW), :], inbs[q % 2], sins[q % 2])

    def out_copy(q):
        si, k = q // NWIN, q % NWIN
        return pltpu.make_async_copy(
            outbs[q % 2], out_hbm.at[s0 + si, pl.ds(ORW * k, ORW), :],
            souts[q % 2])

    in_copy(0).start()
    for q in range(NQ):
        k = q % NWIN
        h0 = 32 * k
        hs = max(32 * k - 8, 0)
        inb = inbs[q % 2]
        outb = outbs[q % 2]

        in_copy(q).wait()
        if q + 1 < NQ:
            in_copy(q + 1).start()
        if q >= 2:
            out_copy(q - 2).wait()

        for g in range(OW // L):
            outb[ORW - 1, pl.ds(g * L, L)] = zf
            if k == 0:
                outb[0, pl.ds(g * L, L)] = zf

        def row(j, _):
            h = h0 - 1 + j
            hv = jnp.broadcast_to(h, (L,))
            grow = jnp.minimum(2 * hv + dyv, hmax)
            rlv = grow - 2 * h0
            m = (hv >= zero) & (rlv >= zero) & (rlv < rows)
            jloc = jnp.maximum(h, 0) - hs
            for g in range(GROUPS):
                x = inb[jloc, pl.ds(g * L, L)]
                plsc.store_scatter(outb, [rlv, colv[g]], x, mask=m)
            return 0
        lax.fori_loop(0, 33, row, 0)

        out_copy(q).start()
    out_copy(NQ - 2).wait()
    out_copy(NQ - 1).wait()


@jax.jit
def _zero_upsample_sc(in3, jit64):
    mesh = plsc.VectorSubcoreMesh(core_axis_name="c", subcore_axis_name="s")
    f = pl.kernel(
        _sc_body,
        out_type=jax.ShapeDtypeStruct((SLICES, OH, OW), jnp.float32),
        mesh=mesh,
        compiler_params=pltpu.CompilerParams(needs_layout_passes=False),
        scratch_types=[
            pltpu.VMEM((4 * L,), jnp.float32),
            pltpu.VMEM((IRW, W), jnp.float32),
            pltpu.VMEM((IRW, W), jnp.float32),
            pltpu.VMEM((ORW, OW), jnp.float32),
            pltpu.VMEM((ORW, OW), jnp.float32),
            pltpu.SemaphoreType.DMA,
            pltpu.SemaphoreType.DMA,
            pltpu.SemaphoreType.DMA,
            pltpu.SemaphoreType.DMA,
        ],
    )
    return f(in3, jit64)


def kernel(ten_in, jitter):
    jit64 = jnp.broadcast_to(
        jitter.astype(jnp.float32).reshape(4, 1), (4, L)).reshape(4 * L)
    out3 = _zero_upsample_sc(ten_in.reshape(SLICES, H, W), jit64)
    return out3.reshape(B, C, OH, OW)


# speedup vs baseline: 417.2648x; 1.1069x over previous
import jax
import jax.numpy as jnp
from jax import lax
from jax.experimental import pallas as pl
from jax.experimental.pallas import tpu as pltpu
from jax.experimental.pallas import tpu_sc as plsc

B, C, H, W = 2, 96, 224, 224
OH, OW = 448, 448
L = 16
NC, NS = 2, 16
NW = NC * NS
SLICES = B * C
SPT = SLICES // NW
IRW = 32
ORW = 64
NWIN = OH // ORW
NQ = SPT * NWIN
GROUPS = W // L


def _sc_body(in_hbm, jit_hbm, out_hbm, jbuf, inb0, inb1, inb2,
             outb0, outb1, s_in0, s_in1, s_in2, s_out0, s_out1):
    wid = lax.axis_index("s") * NC + lax.axis_index("c")
    s0 = wid * SPT
    b = s0 // C

    pltpu.sync_copy(jit_hbm, jbuf)
    jy0 = jbuf[pl.ds(0, L)]
    jx0 = jbuf[pl.ds(L, L)]
    jy1 = jbuf[pl.ds(2 * L, L)]
    jx1 = jbuf[pl.ds(3 * L, L)]
    bv = jnp.broadcast_to(b, (L,))
    jy = jnp.where(bv == 0, jy0, jy1)
    jx = jnp.where(bv == 0, jx0, jx1)
    half = jnp.full((L,), 0.5, jnp.float32)
    one = jnp.full((L,), 1, jnp.int32)
    two = jnp.full((L,), 2, jnp.int32)
    dyv = jnp.where(jy >= half, two, one)
    dxv = jnp.where(jx >= half, two, one)

    iota = lax.iota(jnp.int32, L)
    wmax = jnp.full((L,), OW - 1, jnp.int32)
    r0 = jnp.full((L,), 0, jnp.int32)
    r63 = jnp.full((L,), ORW - 1, jnp.int32)
    dy_is1 = dyv == one
    dy_is2 = dyv == two
    colv = [jnp.minimum(2 * L * g + 2 * iota + dxv, wmax) for g in range(GROUPS)]

    zf = jnp.zeros((L,), jnp.float32)

    for g in range(OW // L):
        def zb(i, _, g=g):
            outb0[i, pl.ds(g * L, L)] = zf
            outb1[i, pl.ds(g * L, L)] = zf
            return 0
        lax.fori_loop(0, ORW, zb, 0)

    inbs = (inb0, inb1, inb2)
    outbs = (outb0, outb1)
    sins = (s_in0, s_in1, s_in2)
    souts = (s_out0, s_out1)

    def plane(si, _):
        def in_copy(k):
            return pltpu.make_async_copy(
                in_hbm.at[s0 + si, pl.ds(IRW * k, IRW), :], inbs[k % 3],
                sins[k % 3])

        def out_copy(k):
            return pltpu.make_async_copy(
                outbs[k % 2], out_hbm.at[s0 + si, pl.ds(ORW * k, ORW), :],
                souts[k % 2])

        in_copy(0).start()
        for k in range(NWIN):
            inb = inbs[k % 3]
            prev = inbs[(k - 1) % 3]
            outb = outbs[k % 2]

            in_copy(k).wait()
            if k + 1 < NWIN:
                in_copy(k + 1).start()
            if k >= 2:
                out_copy(k - 2).wait()

            def rz(g, _, outb=outb, k=k):
                outb[ORW - 1, pl.ds(g * L, L)] = zf
                if k == 0:
                    outb[0, pl.ds(g * L, L)] = zf
                return 0
            lax.fori_loop(0, OW // L, rz, 0)

            if k > 0:
                for g in range(GROUPS):
                    plsc.store_scatter(outb, [r0, colv[g]],
                                       prev[IRW - 1, pl.ds(g * L, L)],
                                       mask=dy_is2)
            m63 = None if k == NWIN - 1 else dy_is1
            for g in range(GROUPS):
                plsc.store_scatter(outb, [r63, colv[g]],
                                   inb[IRW - 1, pl.ds(g * L, L)], mask=m63)

            @plsc.parallel_loop(0, IRW - 1, unroll=4)
            def row(j, inb=inb, outb=outb):
                rlv = 2 * j + dyv
                for g in range(GROUPS):
                    plsc.store_scatter(outb, [rlv, colv[g]],
                                       inb[j, pl.ds(g * L, L)])

            out_copy(k).start()
        out_copy(NWIN - 2).wait()
        out_copy(NWIN - 1).wait()
        return 0

    lax.fori_loop(0, SPT, plane, 0)


@jax.jit
def _zero_upsample_sc(in3, jit64):
    mesh = plsc.VectorSubcoreMesh(core_axis_name="c", subcore_axis_name="s")
    f = pl.kernel(
        _sc_body,
        out_type=jax.ShapeDtypeStruct((SLICES, OH, OW), jnp.float32),
        mesh=mesh,
        compiler_params=pltpu.CompilerParams(needs_layout_passes=False),
        scratch_types=[
            pltpu.VMEM((4 * L,), jnp.float32),
            pltpu.VMEM((IRW, W), jnp.float32),
            pltpu.VMEM((IRW, W), jnp.float32),
            pltpu.VMEM((IRW, W), jnp.float32),
            pltpu.VMEM((ORW, OW), jnp.float32),
            pltpu.VMEM((ORW, OW), jnp.float32),
            pltpu.SemaphoreType.DMA,
            pltpu.SemaphoreType.DMA,
            pltpu.SemaphoreType.DMA,
            pltpu.SemaphoreType.DMA,
            pltpu.SemaphoreType.DMA,
        ],
    )
    return f(in3, jit64)


def kernel(ten_in, jitter):
    jit64 = jnp.broadcast_to(
        jitter.astype(jnp.float32).reshape(4, 1), (4, L)).reshape(4 * L)
    out3 = _zero_upsample_sc(ten_in.reshape(SLICES, H, W), jit64)
    return out3.reshape(B, C, OH, OW)
